# baseline (speedup 1.0000x reference)
import math

import jax
import jax.numpy as jnp
import numpy as np
from jax import lax
from jax.experimental import pallas as pl
from jax.experimental.pallas import tpu as pltpu
from jax.experimental.pallas import tpu_sc as plsc

VOCAB = 1000000
MAX_LEN = 512
DIM = 64
BATCH = 1024
SEQ = 200

NW = 32
SPW = BATCH // NW
HALF = SEQ // 2
NB = 4
NG = SPW // NB

VCHUNK = 384
NCHUNK = (VOCAB - DIM) // VCHUNK
CPW = NCHUNK // NW
PAIRS = VOCAB // 2
TAIL_V = NCHUNK * VCHUNK


def _pos_table():
    den = np.exp(-np.arange(0, DIM, 2, dtype=np.float64) * math.log(10000.0) / DIM)
    pos = np.arange(0, MAX_LEN, dtype=np.float64).reshape(MAX_LEN, 1)
    pe = np.zeros((MAX_LEN, DIM), dtype=np.float64)
    pe[:, 0::2] = np.sin(pos * den)
    pe[:, 1::2] = np.cos(pos * den)
    return jnp.asarray(pe[:SEQ], dtype=jnp.float32)


def _transpose_body(tabt_hbm, tailp_hbm, pairs_hbm, *scratch):
    stg = scratch[0:2]
    outst = scratch[2:4]
    sem_i = scratch[4:6]
    sem_o = scratch[6:8]

    wid = lax.axis_index("s") * 2 + lax.axis_index("c")

    iota = lax.iota(jnp.int32, 16)
    rowj = [lax.shift_right_logical(iota + 16 * j, 1) for j in range(VCHUNK // 16)]
    colj = [((iota + 16 * j) & 1) * DIM for j in range(VCHUNK // 16)]

    def start_stage(c, b):
        pltpu.async_copy(tabt_hbm.at[:, pl.ds(c * VCHUNK, VCHUNK)], stg[b], sem_i[b])

    def wait_stage(c, b):
        pltpu.make_async_copy(
            tabt_hbm.at[:, pl.ds(c * VCHUNK, VCHUNK)], stg[b], sem_i[b]
        ).wait()

    def start_out(c, b):
        pltpu.async_copy(
            outst[b], pairs_hbm.at[pl.ds(c * (VCHUNK // 2), VCHUNK // 2)], sem_o[b]
        )

    def wait_out(c, b):
        pltpu.make_async_copy(
            outst[b], pairs_hbm.at[pl.ds(c * (VCHUNK // 2), VCHUNK // 2)], sem_o[b]
        ).wait()

    def transpose_chunk(b):
        s_ref = stg[b]
        o_ref = outst[b]

        @plsc.parallel_loop(0, DIM, step=1, unroll=4)
        def _drow(d):
            for j in range(VCHUNK // 16):
                vals = s_ref[d, pl.ds(16 * j, 16)]
                plsc.store_scatter(o_ref, [rowj[j], colj[j] + d], vals)

    def step(i, b, prefetch, drain):
        c = wid + NW * i
        if prefetch:
            start_stage(c + NW, 1 - b)
        wait_stage(c, b)
        if drain:
            wait_out(c - 2 * NW, b)
        transpose_chunk(b)
        start_out(c, b)

    start_stage(wid, 0)
    step(0, 0, prefetch=True, drain=False)
    step(1, 1, prefetch=True, drain=False)

    def group(g, carry):
        for b in range(2):
            step(2 * g + b, b, prefetch=True, drain=True)
        return carry

    lax.fori_loop(1, CPW // 2, group, 0)
    step(CPW - 1, (CPW - 1) % 2, prefetch=False, drain=True)
    wait_out(wid + NW * (CPW - 2), (CPW - 2) % 2)
    wait_out(wid + NW * (CPW - 1), (CPW - 1) % 2)

    @pl.when(wid < NCHUNK - CPW * NW)
    def _extra():
        c = CPW * NW + wid
        pltpu.sync_copy(tabt_hbm.at[:, pl.ds(c * VCHUNK, VCHUNK)], stg[0])
        transpose_chunk(0)
        pltpu.sync_copy(outst[0], pairs_hbm.at[pl.ds(c * (VCHUNK // 2), VCHUNK // 2)])

    @pl.when(wid == 12)
    def _copy_tail():
        pltpu.sync_copy(tailp_hbm, pairs_hbm.at[pl.ds(TAIL_V // 2, 32)])


def _gather_body(table_hbm, tok_hbm, pos_hbm, out_hbm, idx_v, pos_v, *bufs_and_sems):
    bufs = bufs_and_sems[:NB]
    sem_g = bufs_and_sems[NB:2 * NB]
    sem_o = bufs_and_sems[2 * NB:3 * NB]

    wid = lax.axis_index("s") * 2 + lax.axis_index("c")
    sbase = wid * SPW

    pltpu.sync_copy(tok_hbm.at[pl.ds(sbase, SPW)], idx_v)
    pltpu.sync_copy(pos_hbm, pos_v)

    def gather_halves(s, b):
        for h in range(2):
            yield pltpu.make_async_copy(
                table_hbm.at[idx_v.at[s, h]],
                bufs[b].at[pl.ds(h * HALF, HALF)],
                sem_g[b],
            )

    def start_gather(s, b):
        for cp in gather_halves(s, b):
            cp.start()

    def wait_gather(s, b):
        for cp in gather_halves(s, b):
            cp.wait()

    def start_out(s, b):
        pltpu.async_copy(bufs[b], out_hbm.at[sbase + s], sem_o[b])

    def wait_out(s, b):
        pltpu.make_async_copy(bufs[b], out_hbm.at[sbase + s], sem_o[b]).wait()

    def add_pos(b):
        buf = bufs[b]

        @plsc.parallel_loop(0, SEQ, step=1, unroll=8)
        def _add(r):
            for t in range(4):
                sl = pl.ds(t * 16, 16)
                buf[r, sl] = buf[r, sl] + pos_v[r, sl]

    for b in range(NB - 1):
        start_gather(b, b)

    def step(s, b, first, issue_ahead=True):
        bn = (b - 1) % NB
        if issue_ahead:
            if not first:
                wait_out(s - 1, bn)
            start_gather(s + NB - 1, bn)
        wait_gather(s, b)
        add_pos(b)
        start_out(s, b)

    for b in range(NB):
        step(b, b, first=(b == 0))

    def group(g, carry):
        for b in range(NB):
            step(g * NB + b, b, first=False)
        return carry

    lax.fori_loop(1, NG - 1, group, 0)
    for b in range(NB):
        s = (NG - 1) * NB + b
        step(s, b, first=False, issue_ahead=(s + NB - 1 < SPW))

    for b in range(NB):
        wait_out((NG - 1) * NB + b, b)


def kernel(tokens, table):
    tabt = table.T
    tailp = table[TAIL_V:].reshape(32, 128)
    tok = tokens.astype(jnp.int32).reshape(BATCH, 2, HALF)
    pos = _pos_table()

    mesh = plsc.VectorSubcoreMesh(core_axis_name="c", subcore_axis_name="s")

    k1 = pl.kernel(
        _transpose_body,
        mesh=mesh,
        compiler_params=pltpu.CompilerParams(needs_layout_passes=False),
        out_type=jax.ShapeDtypeStruct((PAIRS, 128), jnp.float32),
        scratch_types=(
            [pltpu.VMEM((DIM, VCHUNK), jnp.float32) for _ in range(2)]
            + [pltpu.VMEM((VCHUNK // 2, 128), jnp.float32) for _ in range(2)]
            + [pltpu.SemaphoreType.DMA for _ in range(4)]
        ),
    )
    pairs = k1(tabt, tailp)
    tab_lin = pairs.reshape(VOCAB, DIM)

    k2 = pl.kernel(
        _gather_body,
        mesh=mesh,
        compiler_params=pltpu.CompilerParams(use_tc_tiling_on_sc=False),
        out_type=jax.ShapeDtypeStruct((BATCH, SEQ, DIM), jnp.float32),
        scratch_types=(
            [pltpu.VMEM((SPW, 2, HALF), jnp.int32),
             pltpu.VMEM((SEQ, DIM), jnp.float32)]
            + [pltpu.VMEM((SEQ, DIM), jnp.float32) for _ in range(NB)]
            + [pltpu.SemaphoreType.DMA for _ in range(2 * NB)]
        ),
    )
    return k2(tab_lin, tok, pos)


# speedup vs baseline: 1.3519x; 1.3519x over previous
import math

import jax
import jax.numpy as jnp
import numpy as np
from jax import lax
from jax.experimental import pallas as pl
from jax.experimental.pallas import tpu as pltpu
from jax.experimental.pallas import tpu_sc as plsc

VOCAB = 1000000
MAX_LEN = 512
DIM = 64
BATCH = 1024
SEQ = 200

NW = 32
SPW = BATCH // NW
HALF = SEQ // 2
NB = 4
NG = SPW // NB

def _pos_table():
    den = np.exp(-np.arange(0, DIM, 2, dtype=np.float64) * math.log(10000.0) / DIM)
    pos = np.arange(0, SEQ, dtype=np.float64).reshape(SEQ, 1)
    pe = np.zeros((SEQ, DIM), dtype=np.float64)
    pe[:, 0::2] = np.sin(pos * den)
    pe[:, 1::2] = np.cos(pos * den)
    return jnp.asarray(pe, dtype=jnp.float32)


def _gather_body(table_hbm, tok_hbm, pos_hbm, out_hbm, idx_v, pos_v, *bufs_and_sems):
    bufs = bufs_and_sems[:NB]
    sem_g = bufs_and_sems[NB:2 * NB]
    sem_o = bufs_and_sems[2 * NB:3 * NB]

    wid = lax.axis_index("s") * 2 + lax.axis_index("c")
    sbase = wid * SPW

    pltpu.sync_copy(tok_hbm.at[pl.ds(sbase, SPW)], idx_v)
    pltpu.sync_copy(pos_hbm, pos_v)

    def gather_halves(s, b):
        for h in range(2):
            yield pltpu.make_async_copy(
                table_hbm.at[idx_v.at[s, h]],
                bufs[b].at[pl.ds(h * HALF, HALF)],
                sem_g[b],
            )

    def start_gather(s, b):
        for cp in gather_halves(s, b):
            cp.start()

    def wait_gather(s, b):
        for cp in gather_halves(s, b):
            cp.wait()

    def start_out(s, b):
        pltpu.async_copy(bufs[b], out_hbm.at[sbase + s], sem_o[b])

    def wait_out(s, b):
        pltpu.make_async_copy(bufs[b], out_hbm.at[sbase + s], sem_o[b]).wait()

    def add_pos(b):
        buf = bufs[b]

        @plsc.parallel_loop(0, SEQ, step=1, unroll=8)
        def _add(r):
            for t in range(4):
                sl = pl.ds(t * 16, 16)
                buf[r, sl] = buf[r, sl] + pos_v[r, sl]

    for b in range(NB - 1):
        start_gather(b, b)

    def step(s, b, first, issue_ahead=True):
        bn = (b - 1) % NB
        if issue_ahead:
            if not first:
                wait_out(s - 1, bn)
            start_gather(s + NB - 1, bn)
        wait_gather(s, b)
        add_pos(b)
        start_out(s, b)

    for b in range(NB):
        step(b, b, first=(b == 0))

    def group(g, carry):
        for b in range(NB):
            step(g * NB + b, b, first=False)
        return carry

    lax.fori_loop(1, NG - 1, group, 0)
    for b in range(NB):
        s = (NG - 1) * NB + b
        step(s, b, first=False, issue_ahead=(s + NB - 1 < SPW))

    for b in range(NB):
        wait_out((NG - 1) * NB + b, b)


def kernel(tokens, table):
    tok = tokens.astype(jnp.int32).reshape(BATCH, 2, HALF)
    pos = _pos_table()

    mesh = plsc.VectorSubcoreMesh(core_axis_name="c", subcore_axis_name="s")
    run = pl.kernel(
        _gather_body,
        mesh=mesh,
        compiler_params=pltpu.CompilerParams(use_tc_tiling_on_sc=False),
        out_type=jax.ShapeDtypeStruct((BATCH, SEQ, DIM), jnp.float32),
        scratch_types=(
            [pltpu.VMEM((SPW, 2, HALF), jnp.int32),
             pltpu.VMEM((SEQ, DIM), jnp.float32)]
            + [pltpu.VMEM((SEQ, DIM), jnp.float32) for _ in range(NB)]
            + [pltpu.SemaphoreType.DMA for _ in range(2 * NB)]
        ),
    )
    return run(table, tok, pos)
